# Initial kernel scaffold; baseline (speedup 1.0000x reference)
import jax
import jax.numpy as jnp
from jax.experimental import pallas as pl


def kernel(x, W_in, b_in, msgW1, msgb1, msgW2, msgb2, updW1, updb1, updW2, updb2, gamma, beta, O1, o1, O2, o2, edge_index, n_qubits):
    raise NotImplementedError("write your pallas kernel here")


# speedup vs baseline: 5.3966x; 5.3966x over previous
import functools

import jax
import jax.numpy as jnp
from jax import lax
from jax.experimental import pallas as pl
from jax.experimental.pallas import tpu as pltpu
from jax.experimental.pallas import tpu_sc as plsc

_N = 50000
_E = 800000
_H = 64
_NQ = 25000
_L = 5

_BR = 2000
_BQ = 5000


def _inproj_body(x_ref, w_ref, b_ref, o_ref):
    o_ref[...] = jnp.maximum(
        jnp.dot(x_ref[...], w_ref[...], preferred_element_type=jnp.float32)
        + b_ref[...], 0.0)


def _in_proj(x, W_in, b_in):
    return pl.pallas_call(
        _inproj_body,
        grid=(_N // _BR,),
        in_specs=[
            pl.BlockSpec((_BR, 3), lambda i: (i, 0)),
            pl.BlockSpec((3, _H), lambda i: (0, 0)),
            pl.BlockSpec((1, _H), lambda i: (0, 0)),
        ],
        out_specs=pl.BlockSpec((_BR, _H), lambda i: (i, 0)),
        out_shape=jax.ShapeDtypeStruct((_N, _H), jnp.float32),
    )(x, W_in, b_in.reshape(1, _H))


def _msg_body(h_ref, w1_ref, b1_ref, w2_ref, b2_ref, m0_ref, m1_ref):
    t = jnp.maximum(
        jnp.dot(h_ref[...], w1_ref[...], preferred_element_type=jnp.float32)
        + b1_ref[...], 0.0)
    m = jnp.dot(t, w2_ref[...], preferred_element_type=jnp.float32) + b2_ref[...]
    m0_ref[...] = m[:, : _H // 2]
    m1_ref[...] = m[:, _H // 2:]


def _msg(h, w1, b1, w2, b2):
    return pl.pallas_call(
        _msg_body,
        grid=(_N // _BR,),
        in_specs=[
            pl.BlockSpec((_BR, _H), lambda i: (i, 0)),
            pl.BlockSpec((_H, _H), lambda i: (0, 0)),
            pl.BlockSpec((1, _H), lambda i: (0, 0)),
            pl.BlockSpec((_H, _H), lambda i: (0, 0)),
            pl.BlockSpec((1, _H), lambda i: (0, 0)),
        ],
        out_specs=[
            pl.BlockSpec((_BR, _H // 2), lambda i: (i, 0)),
            pl.BlockSpec((_BR, _H // 2), lambda i: (i, 0)),
        ],
        out_shape=[
            jax.ShapeDtypeStruct((_N, _H // 2), jnp.float32),
            jax.ShapeDtypeStruct((_N, _H // 2), jnp.float32),
        ],
    )(h, w1, b1.reshape(1, _H), w2, b2.reshape(1, _H))


def _upd_body(h_ref, a0_ref, a1_ref, w1_ref, b1_ref, w2_ref, b2_ref,
              o_ref, st_ref):
    comb = jnp.concatenate([h_ref[...], a0_ref[...], a1_ref[...]], axis=1)
    t = jnp.maximum(
        jnp.dot(comb, w1_ref[...], preferred_element_type=jnp.float32)
        + b1_ref[...], 0.0)
    o = jnp.dot(t, w2_ref[...], preferred_element_type=jnp.float32) + b2_ref[...]
    o_ref[...] = o
    s = jnp.sum(o, axis=0, keepdims=True)
    s2 = jnp.sum(o * o, axis=0, keepdims=True)
    st = jnp.concatenate([s, s2], axis=0)

    @pl.when(pl.program_id(0) == 0)
    def _():
        st_ref[...] = st

    @pl.when(pl.program_id(0) != 0)
    def _():
        st_ref[...] += st


def _upd(h, a0, a1, w1, b1, w2, b2):
    return pl.pallas_call(
        _upd_body,
        grid=(_N // _BR,),
        in_specs=[
            pl.BlockSpec((_BR, _H), lambda i: (i, 0)),
            pl.BlockSpec((_BR, _H // 2), lambda i: (i, 0)),
            pl.BlockSpec((_BR, _H // 2), lambda i: (i, 0)),
            pl.BlockSpec((2 * _H, _H), lambda i: (0, 0)),
            pl.BlockSpec((1, _H), lambda i: (0, 0)),
            pl.BlockSpec((_H, _H), lambda i: (0, 0)),
            pl.BlockSpec((1, _H), lambda i: (0, 0)),
        ],
        out_specs=[
            pl.BlockSpec((_BR, _H), lambda i: (i, 0)),
            pl.BlockSpec((2, _H), lambda i: (0, 0)),
        ],
        out_shape=[
            jax.ShapeDtypeStruct((_N, _H), jnp.float32),
            jax.ShapeDtypeStruct((2, _H), jnp.float32),
        ],
    )(h, a0, a1, w1, b1.reshape(1, _H), w2, b2.reshape(1, _H))


def _norm_body(o_ref, h_ref, st_ref, g_ref, b_ref, out_ref):
    mean = st_ref[0:1, :] * (1.0 / _N)
    var = st_ref[1:2, :] * (1.0 / _N) - mean * mean
    inv = lax.rsqrt(var + 1e-5)
    out_ref[...] = jnp.maximum(
        (o_ref[...] - mean) * inv * g_ref[...] + b_ref[...] + h_ref[...], 0.0)


def _norm(o, h, st, g, b):
    return pl.pallas_call(
        _norm_body,
        grid=(_N // _BR,),
        in_specs=[
            pl.BlockSpec((_BR, _H), lambda i: (i, 0)),
            pl.BlockSpec((_BR, _H), lambda i: (i, 0)),
            pl.BlockSpec((2, _H), lambda i: (0, 0)),
            pl.BlockSpec((1, _H), lambda i: (0, 0)),
            pl.BlockSpec((1, _H), lambda i: (0, 0)),
        ],
        out_specs=pl.BlockSpec((_BR, _H), lambda i: (i, 0)),
        out_shape=jax.ShapeDtypeStruct((_N, _H), jnp.float32),
    )(o, h, st, g.reshape(1, _H), b.reshape(1, _H))


def _read_body(q_ref, o1_ref, b1_ref, o2_ref, b2_ref, out_ref):
    t = jnp.maximum(
        jnp.dot(q_ref[...], o1_ref[...], preferred_element_type=jnp.float32)
        + b1_ref[...], 0.0)
    out_ref[...] = (
        jnp.dot(t, o2_ref[...], preferred_element_type=jnp.float32) + b2_ref[...])


def _read(q, O1, o1, O2, o2):
    return pl.pallas_call(
        _read_body,
        grid=(_NQ // _BQ,),
        in_specs=[
            pl.BlockSpec((_BQ, _H), lambda i: (i, 0)),
            pl.BlockSpec((_H, _H // 2), lambda i: (0, 0)),
            pl.BlockSpec((1, _H // 2), lambda i: (0, 0)),
            pl.BlockSpec((_H // 2, 1), lambda i: (0, 0)),
            pl.BlockSpec((1, 1), lambda i: (0, 0)),
        ],
        out_specs=pl.BlockSpec((_BQ, 1), lambda i: (i, 0)),
        out_shape=jax.ShapeDtypeStruct((_NQ, 1), jnp.float32),
    )(q, O1, o1.reshape(1, _H // 2), O2, o2.reshape(1, 1))


_NSUB = 16
_S0 = 3128
_S15 = _N - 15 * _S0
_EPT = _E // _NSUB
_IB = 125
_JN = 16
_CH = _IB * _JN
_NCH = _EPT // _CH
_HC = _H // 2

_sc_mesh = plsc.VectorSubcoreMesh(core_axis_name="c", subcore_axis_name="s")


@functools.partial(
    pl.kernel,
    out_type=jax.ShapeDtypeStruct((2 * _N, _HC), jnp.float32),
    mesh=_sc_mesh,
    compiler_params=pltpu.CompilerParams(use_tc_tiling_on_sc=False),
    scratch_types=[
        pltpu.VMEM((_JN, _IB), jnp.int32),
        pltpu.VMEM((_JN, _IB), jnp.int32),
        pltpu.VMEM((_IB, _HC), jnp.float32),
        pltpu.VMEM_SHARED((_N, _HC), jnp.float32),
        pltpu.SemaphoreType.DMA,
    ],
)
def _sc_scatter(m0, m1, src2d, dst2d, zblk, out, src_v, dst_v, rows_v, acc, sem):
    cid = lax.axis_index("c")
    sid = lax.axis_index("s")

    @pl.when(sid < _NSUB - 1)
    def _():
        pltpu.sync_copy(zblk.at[pl.ds(0, _S0)], acc.at[pl.ds(sid * _S0, _S0)])

    @pl.when(sid == _NSUB - 1)
    def _():
        pltpu.sync_copy(zblk.at[pl.ds(0, _S15)],
                        acc.at[pl.ds((_NSUB - 1) * _S0, _S15)])

    plsc.subcore_barrier()

    def run(m):
        def body(ci, carry):
            rb = sid * (_EPT // _IB) + ci * _JN
            pltpu.sync_copy(src2d.at[pl.ds(rb, _JN)], src_v)
            pltpu.sync_copy(dst2d.at[pl.ds(rb, _JN)], dst_v)
            for j in range(_JN):
                pltpu.async_copy(m.at[src_v.at[j]], rows_v, sem).wait()
                pltpu.sync_copy(rows_v, acc.at[dst_v.at[j]], add=True)
            return carry
        lax.fori_loop(0, _NCH, body, 0)

    @pl.when(cid == 0)
    def _():
        run(m0)

    @pl.when(cid == 1)
    def _():
        run(m1)

    plsc.subcore_barrier()

    @pl.when(sid < _NSUB - 1)
    def _():
        pltpu.sync_copy(
            acc.at[pl.ds(sid * _S0, _S0)],
            out.at[pl.ds(cid * _N + sid * _S0, _S0)],
        )

    @pl.when(sid == _NSUB - 1)
    def _():
        pltpu.sync_copy(
            acc.at[pl.ds((_NSUB - 1) * _S0, _S15)],
            out.at[pl.ds(cid * _N + (_NSUB - 1) * _S0, _S15)],
        )


def kernel(x, W_in, b_in, msgW1, msgb1, msgW2, msgb2, updW1, updb1, updW2,
           updb2, gamma, beta, O1, o1, O2, o2, edge_index, n_qubits):
    src2d = edge_index[0].reshape(_E // _IB, _IB)
    dst2d = edge_index[1].reshape(_E // _IB, _IB)
    zblk = jnp.zeros((_S0, _HC), jnp.float32)
    h = _in_proj(x, W_in, b_in)
    for i in range(_L):
        m0, m1 = _msg(h, msgW1[i], msgb1[i], msgW2[i], msgb2[i])
        agg = _sc_scatter(m0, m1, src2d, dst2d, zblk)
        out_pre, st = _upd(h, agg[:_N], agg[_N:], updW1[i], updb1[i],
                           updW2[i], updb2[i])
        h = _norm(out_pre, h, st, gamma[i], beta[i])
    q = lax.dynamic_slice_in_dim(h, n_qubits - _NQ, _NQ, axis=0)
    return _read(q, O1, o1, O2, o2)


# speedup vs baseline: 8.5777x; 1.5894x over previous
import functools

import jax
import jax.numpy as jnp
from jax import lax
from jax.experimental import pallas as pl
from jax.experimental.pallas import tpu as pltpu
from jax.experimental.pallas import tpu_sc as plsc

_N = 50000
_E = 800000
_H = 64
_NQ = 25000
_L = 5

_BR = 2000
_BQ = 5000


def _rbf16(a):
    u = lax.bitcast_convert_type(a, jnp.uint32)
    u = u + jnp.uint32(0x7FFF) + ((u >> jnp.uint32(16)) & jnp.uint32(1))
    u = u & jnp.uint32(0xFFFF0000)
    return lax.bitcast_convert_type(u, jnp.float32)


def _dot(a, b):
    return jnp.dot(_rbf16(a), _rbf16(b), preferred_element_type=jnp.float32)


def _inproj_body(x_ref, w_ref, b_ref, o_ref):
    o_ref[...] = jnp.maximum(
        _dot(x_ref[...], w_ref[...])
        + b_ref[...], 0.0)


def _in_proj(x, W_in, b_in):
    return pl.pallas_call(
        _inproj_body,
        grid=(_N // _BR,),
        in_specs=[
            pl.BlockSpec((_BR, 3), lambda i: (i, 0)),
            pl.BlockSpec((3, _H), lambda i: (0, 0)),
            pl.BlockSpec((1, _H), lambda i: (0, 0)),
        ],
        out_specs=pl.BlockSpec((_BR, _H), lambda i: (i, 0)),
        out_shape=jax.ShapeDtypeStruct((_N, _H), jnp.float32),
    )(x, W_in, b_in.reshape(1, _H))


def _msg_body(h_ref, w1_ref, b1_ref, w2_ref, b2_ref, m0_ref, m1_ref):
    t = jnp.maximum(
        _dot(h_ref[...], w1_ref[...])
        + b1_ref[...], 0.0)
    m = _dot(t, w2_ref[...]) + b2_ref[...]
    m0_ref[...] = m[:, : _H // 2]
    m1_ref[...] = m[:, _H // 2:]


def _msg(h, w1, b1, w2, b2):
    return pl.pallas_call(
        _msg_body,
        grid=(_N // _BR,),
        in_specs=[
            pl.BlockSpec((_BR, _H), lambda i: (i, 0)),
            pl.BlockSpec((_H, _H), lambda i: (0, 0)),
            pl.BlockSpec((1, _H), lambda i: (0, 0)),
            pl.BlockSpec((_H, _H), lambda i: (0, 0)),
            pl.BlockSpec((1, _H), lambda i: (0, 0)),
        ],
        out_specs=[
            pl.BlockSpec((_BR, _H // 2), lambda i: (i, 0)),
            pl.BlockSpec((_BR, _H // 2), lambda i: (i, 0)),
        ],
        out_shape=[
            jax.ShapeDtypeStruct((_N, _H // 2), jnp.float32),
            jax.ShapeDtypeStruct((_N, _H // 2), jnp.float32),
        ],
    )(h, w1, b1.reshape(1, _H), w2, b2.reshape(1, _H))


def _upd_body(h_ref, a0_ref, a1_ref, w1_ref, b1_ref, w2_ref, b2_ref,
              o_ref, st_ref):
    comb = jnp.concatenate([h_ref[...], a0_ref[...], a1_ref[...]], axis=1)
    t = jnp.maximum(
        _dot(comb, w1_ref[...])
        + b1_ref[...], 0.0)
    o = _dot(t, w2_ref[...]) + b2_ref[...]
    o_ref[...] = o
    s = jnp.sum(o, axis=0, keepdims=True)
    s2 = jnp.sum(o * o, axis=0, keepdims=True)
    st = jnp.concatenate([s, s2], axis=0)

    @pl.when(pl.program_id(0) == 0)
    def _():
        st_ref[...] = st

    @pl.when(pl.program_id(0) != 0)
    def _():
        st_ref[...] += st


def _upd(h, a0, a1, w1, b1, w2, b2):
    return pl.pallas_call(
        _upd_body,
        grid=(_N // _BR,),
        in_specs=[
            pl.BlockSpec((_BR, _H), lambda i: (i, 0)),
            pl.BlockSpec((_BR, _H // 2), lambda i: (i, 0)),
            pl.BlockSpec((_BR, _H // 2), lambda i: (i, 0)),
            pl.BlockSpec((2 * _H, _H), lambda i: (0, 0)),
            pl.BlockSpec((1, _H), lambda i: (0, 0)),
            pl.BlockSpec((_H, _H), lambda i: (0, 0)),
            pl.BlockSpec((1, _H), lambda i: (0, 0)),
        ],
        out_specs=[
            pl.BlockSpec((_BR, _H), lambda i: (i, 0)),
            pl.BlockSpec((2, _H), lambda i: (0, 0)),
        ],
        out_shape=[
            jax.ShapeDtypeStruct((_N, _H), jnp.float32),
            jax.ShapeDtypeStruct((2, _H), jnp.float32),
        ],
    )(h, a0, a1, w1, b1.reshape(1, _H), w2, b2.reshape(1, _H))


def _norm_body(o_ref, h_ref, st_ref, g_ref, b_ref, out_ref):
    mean = st_ref[0:1, :] * (1.0 / _N)
    var = st_ref[1:2, :] * (1.0 / _N) - mean * mean
    inv = lax.rsqrt(var + 1e-5)
    out_ref[...] = jnp.maximum(
        (o_ref[...] - mean) * inv * g_ref[...] + b_ref[...] + h_ref[...], 0.0)


def _norm(o, h, st, g, b):
    return pl.pallas_call(
        _norm_body,
        grid=(_N // _BR,),
        in_specs=[
            pl.BlockSpec((_BR, _H), lambda i: (i, 0)),
            pl.BlockSpec((_BR, _H), lambda i: (i, 0)),
            pl.BlockSpec((2, _H), lambda i: (0, 0)),
            pl.BlockSpec((1, _H), lambda i: (0, 0)),
            pl.BlockSpec((1, _H), lambda i: (0, 0)),
        ],
        out_specs=pl.BlockSpec((_BR, _H), lambda i: (i, 0)),
        out_shape=jax.ShapeDtypeStruct((_N, _H), jnp.float32),
    )(o, h, st, g.reshape(1, _H), b.reshape(1, _H))


def _read_body(q_ref, o1_ref, b1_ref, o2_ref, b2_ref, out_ref):
    t = jnp.maximum(
        _dot(q_ref[...], o1_ref[...])
        + b1_ref[...], 0.0)
    out_ref[...] = (
        _dot(t, o2_ref[...]) + b2_ref[...])


def _read(q, O1, o1, O2, o2):
    return pl.pallas_call(
        _read_body,
        grid=(_NQ // _BQ,),
        in_specs=[
            pl.BlockSpec((_BQ, _H), lambda i: (i, 0)),
            pl.BlockSpec((_H, _H // 2), lambda i: (0, 0)),
            pl.BlockSpec((1, _H // 2), lambda i: (0, 0)),
            pl.BlockSpec((_H // 2, 1), lambda i: (0, 0)),
            pl.BlockSpec((1, 1), lambda i: (0, 0)),
        ],
        out_specs=pl.BlockSpec((_BQ, 1), lambda i: (i, 0)),
        out_shape=jax.ShapeDtypeStruct((_NQ, 1), jnp.float32),
    )(q, O1, o1.reshape(1, _H // 2), O2, o2.reshape(1, 1))


_NSUB = 16
_S0 = 3128
_S15 = _N - 15 * _S0
_EPT = _E // _NSUB
_IB = 125
_JN = 16
_CH = _IB * _JN
_NCH = _EPT // _CH
_HC = _H // 2
_NSLOT = 4

_sc_mesh = plsc.VectorSubcoreMesh(core_axis_name="c", subcore_axis_name="s")


@functools.partial(
    pl.kernel,
    out_type=jax.ShapeDtypeStruct((2 * _N, _HC), jnp.float32),
    mesh=_sc_mesh,
    compiler_params=pltpu.CompilerParams(use_tc_tiling_on_sc=False),
    scratch_types=[
        pltpu.VMEM((_JN, _IB), jnp.int32),
        pltpu.VMEM((_JN, _IB), jnp.int32),
        pltpu.VMEM((_NSLOT, _IB, _HC), jnp.float32),
        pltpu.VMEM_SHARED((_N, _HC), jnp.float32),
        pltpu.SemaphoreType.DMA((_NSLOT,)),
        pltpu.SemaphoreType.DMA((_NSLOT,)),
    ],
)
def _sc_scatter(m0, m1, src2d, dst2d, zblk, out, src_v, dst_v, rows_v, acc,
                gsem, ssem):
    cid = lax.axis_index("c")
    sid = lax.axis_index("s")

    @pl.when(sid < _NSUB - 1)
    def _():
        pltpu.sync_copy(zblk.at[pl.ds(0, _S0)], acc.at[pl.ds(sid * _S0, _S0)])

    @pl.when(sid == _NSUB - 1)
    def _():
        pltpu.sync_copy(zblk.at[pl.ds(0, _S15)],
                        acc.at[pl.ds((_NSUB - 1) * _S0, _S15)])

    plsc.subcore_barrier()

    def run(m):
        def body(ci, carry):
            rb = sid * (_EPT // _IB) + ci * _JN
            pltpu.sync_copy(src2d.at[pl.ds(rb, _JN)], src_v)
            pltpu.sync_copy(dst2d.at[pl.ds(rb, _JN)], dst_v)
            gd = [None] * _JN
            sd = [None] * _JN

            def fire_scatter(jj):
                gd[jj].wait()
                sd[jj] = pltpu.async_copy(
                    rows_v.at[jj % _NSLOT], acc.at[dst_v.at[jj]],
                    ssem.at[jj % _NSLOT], add=True)

            for j in range(_JN):
                slot = j % _NSLOT
                if j >= _NSLOT:
                    sd[j - _NSLOT].wait()
                gd[j] = pltpu.async_copy(
                    m.at[src_v.at[j]], rows_v.at[slot], gsem.at[slot])
                if j >= _NSLOT - 1:
                    fire_scatter(j - (_NSLOT - 1))
            for jj in range(_JN - (_NSLOT - 1), _JN):
                fire_scatter(jj)
            for jj in range(_JN - _NSLOT, _JN):
                sd[jj].wait()
            return carry
        lax.fori_loop(0, _NCH, body, 0)

    @pl.when(cid == 0)
    def _():
        run(m0)

    @pl.when(cid == 1)
    def _():
        run(m1)

    plsc.subcore_barrier()

    @pl.when(sid < _NSUB - 1)
    def _():
        pltpu.sync_copy(
            acc.at[pl.ds(sid * _S0, _S0)],
            out.at[pl.ds(cid * _N + sid * _S0, _S0)],
        )

    @pl.when(sid == _NSUB - 1)
    def _():
        pltpu.sync_copy(
            acc.at[pl.ds((_NSUB - 1) * _S0, _S15)],
            out.at[pl.ds(cid * _N + (_NSUB - 1) * _S0, _S15)],
        )


def kernel(x, W_in, b_in, msgW1, msgb1, msgW2, msgb2, updW1, updb1, updW2,
           updb2, gamma, beta, O1, o1, O2, o2, edge_index, n_qubits):
    src2d = edge_index[0].reshape(_E // _IB, _IB)
    dst2d = edge_index[1].reshape(_E // _IB, _IB)
    zblk = jnp.zeros((_S0, _HC), jnp.float32)
    h = _in_proj(x, W_in, b_in)
    for i in range(_L):
        m0, m1 = _msg(h, msgW1[i], msgb1[i], msgW2[i], msgb2[i])
        agg = _sc_scatter(m0, m1, src2d, dst2d, zblk)
        out_pre, st = _upd(h, agg[:_N], agg[_N:], updW1[i], updb1[i],
                           updW2[i], updb2[i])
        h = _norm(out_pre, h, st, gamma[i], beta[i])
    q = lax.dynamic_slice_in_dim(h, n_qubits - _NQ, _NQ, axis=0)
    return _read(q, O1, o1, O2, o2)


# speedup vs baseline: 9.1201x; 1.0632x over previous
import functools

import jax
import jax.numpy as jnp
from jax import lax
from jax.experimental import pallas as pl
from jax.experimental.pallas import tpu as pltpu
from jax.experimental.pallas import tpu_sc as plsc

_N = 50000
_E = 800000
_H = 64
_NQ = 25000
_L = 5

_BR = 2000
_BQ = 5000


def _rbf16(a):
    u = lax.bitcast_convert_type(a, jnp.uint32)
    u = u + jnp.uint32(0x7FFF) + ((u >> jnp.uint32(16)) & jnp.uint32(1))
    u = u & jnp.uint32(0xFFFF0000)
    return lax.bitcast_convert_type(u, jnp.float32)


def _dot(a, b):
    return jnp.dot(_rbf16(a), _rbf16(b), preferred_element_type=jnp.float32)


def _inproj_body(x_ref, w_ref, b_ref, o_ref):
    o_ref[...] = jnp.maximum(
        _dot(x_ref[...], w_ref[...])
        + b_ref[...], 0.0)


def _in_proj(x, W_in, b_in):
    return pl.pallas_call(
        _inproj_body,
        grid=(_N // _BR,),
        in_specs=[
            pl.BlockSpec((_BR, 3), lambda i: (i, 0)),
            pl.BlockSpec((3, _H), lambda i: (0, 0)),
            pl.BlockSpec((1, _H), lambda i: (0, 0)),
        ],
        out_specs=pl.BlockSpec((_BR, _H), lambda i: (i, 0)),
        out_shape=jax.ShapeDtypeStruct((_N, _H), jnp.float32),
    )(x, W_in, b_in.reshape(1, _H))


def _msg_body(h_ref, w1_ref, b1_ref, w2_ref, b2_ref, m0_ref, m1_ref):
    t = jnp.maximum(
        _dot(h_ref[...], w1_ref[...])
        + b1_ref[...], 0.0)
    m = _dot(t, w2_ref[...]) + b2_ref[...]
    m0_ref[...] = m[:, : _H // 2]
    m1_ref[...] = m[:, _H // 2:]


def _msg(h, w1, b1, w2, b2):
    return pl.pallas_call(
        _msg_body,
        grid=(_N // _BR,),
        in_specs=[
            pl.BlockSpec((_BR, _H), lambda i: (i, 0)),
            pl.BlockSpec((_H, _H), lambda i: (0, 0)),
            pl.BlockSpec((1, _H), lambda i: (0, 0)),
            pl.BlockSpec((_H, _H), lambda i: (0, 0)),
            pl.BlockSpec((1, _H), lambda i: (0, 0)),
        ],
        out_specs=[
            pl.BlockSpec((_BR, _H // 2), lambda i: (i, 0)),
            pl.BlockSpec((_BR, _H // 2), lambda i: (i, 0)),
        ],
        out_shape=[
            jax.ShapeDtypeStruct((_N, _H // 2), jnp.float32),
            jax.ShapeDtypeStruct((_N, _H // 2), jnp.float32),
        ],
    )(h, w1, b1.reshape(1, _H), w2, b2.reshape(1, _H))


def _upd_body(h_ref, a0_ref, a1_ref, w1_ref, b1_ref, w2_ref, b2_ref,
              o_ref, st_ref):
    comb = jnp.concatenate([h_ref[...], a0_ref[...], a1_ref[...]], axis=1)
    t = jnp.maximum(
        _dot(comb, w1_ref[...])
        + b1_ref[...], 0.0)
    o = _dot(t, w2_ref[...]) + b2_ref[...]
    o_ref[...] = o
    s = jnp.sum(o, axis=0, keepdims=True)
    s2 = jnp.sum(o * o, axis=0, keepdims=True)
    st = jnp.concatenate([s, s2], axis=0)

    @pl.when(pl.program_id(0) == 0)
    def _():
        st_ref[...] = st

    @pl.when(pl.program_id(0) != 0)
    def _():
        st_ref[...] += st


def _upd(h, a0, a1, w1, b1, w2, b2):
    return pl.pallas_call(
        _upd_body,
        grid=(_N // _BR,),
        in_specs=[
            pl.BlockSpec((_BR, _H), lambda i: (i, 0)),
            pl.BlockSpec((_BR, _H // 2), lambda i: (i, 0)),
            pl.BlockSpec((_BR, _H // 2), lambda i: (i, 0)),
            pl.BlockSpec((2 * _H, _H), lambda i: (0, 0)),
            pl.BlockSpec((1, _H), lambda i: (0, 0)),
            pl.BlockSpec((_H, _H), lambda i: (0, 0)),
            pl.BlockSpec((1, _H), lambda i: (0, 0)),
        ],
        out_specs=[
            pl.BlockSpec((_BR, _H), lambda i: (i, 0)),
            pl.BlockSpec((2, _H), lambda i: (0, 0)),
        ],
        out_shape=[
            jax.ShapeDtypeStruct((_N, _H), jnp.float32),
            jax.ShapeDtypeStruct((2, _H), jnp.float32),
        ],
    )(h, a0, a1, w1, b1.reshape(1, _H), w2, b2.reshape(1, _H))


def _norm_body(o_ref, h_ref, st_ref, g_ref, b_ref, out_ref):
    mean = st_ref[0:1, :] * (1.0 / _N)
    var = st_ref[1:2, :] * (1.0 / _N) - mean * mean
    inv = lax.rsqrt(var + 1e-5)
    out_ref[...] = jnp.maximum(
        (o_ref[...] - mean) * inv * g_ref[...] + b_ref[...] + h_ref[...], 0.0)


def _norm(o, h, st, g, b):
    return pl.pallas_call(
        _norm_body,
        grid=(_N // _BR,),
        in_specs=[
            pl.BlockSpec((_BR, _H), lambda i: (i, 0)),
            pl.BlockSpec((_BR, _H), lambda i: (i, 0)),
            pl.BlockSpec((2, _H), lambda i: (0, 0)),
            pl.BlockSpec((1, _H), lambda i: (0, 0)),
            pl.BlockSpec((1, _H), lambda i: (0, 0)),
        ],
        out_specs=pl.BlockSpec((_BR, _H), lambda i: (i, 0)),
        out_shape=jax.ShapeDtypeStruct((_N, _H), jnp.float32),
    )(o, h, st, g.reshape(1, _H), b.reshape(1, _H))


def _read_body(q_ref, o1_ref, b1_ref, o2_ref, b2_ref, out_ref):
    t = jnp.maximum(
        _dot(q_ref[...], o1_ref[...])
        + b1_ref[...], 0.0)
    out_ref[...] = (
        _dot(t, o2_ref[...]) + b2_ref[...])


def _read(q, O1, o1, O2, o2):
    return pl.pallas_call(
        _read_body,
        grid=(_NQ // _BQ,),
        in_specs=[
            pl.BlockSpec((_BQ, _H), lambda i: (i, 0)),
            pl.BlockSpec((_H, _H // 2), lambda i: (0, 0)),
            pl.BlockSpec((1, _H // 2), lambda i: (0, 0)),
            pl.BlockSpec((_H // 2, 1), lambda i: (0, 0)),
            pl.BlockSpec((1, 1), lambda i: (0, 0)),
        ],
        out_specs=pl.BlockSpec((_BQ, 1), lambda i: (i, 0)),
        out_shape=jax.ShapeDtypeStruct((_NQ, 1), jnp.float32),
    )(q, O1, o1.reshape(1, _H // 2), O2, o2.reshape(1, 1))


_NSUB = 16
_S0 = 3128
_S15 = _N - 15 * _S0
_EPT = _E // _NSUB
_IB = 125
_JN = 16
_CH = _IB * _JN
_NCH = _EPT // _CH
_HC = _H // 2
_NSLOT = 4

_sc_mesh = plsc.VectorSubcoreMesh(core_axis_name="c", subcore_axis_name="s")


@functools.partial(
    pl.kernel,
    out_type=jax.ShapeDtypeStruct((2 * _N, _HC), jnp.float32),
    mesh=_sc_mesh,
    compiler_params=pltpu.CompilerParams(use_tc_tiling_on_sc=False),
    scratch_types=[
        pltpu.VMEM((2, _JN, _IB), jnp.int32),
        pltpu.VMEM((2, _JN, _IB), jnp.int32),
        pltpu.VMEM((_NSLOT, _IB, _HC), jnp.float32),
        pltpu.VMEM_SHARED((_N, _HC), jnp.float32),
        pltpu.SemaphoreType.DMA((_NSLOT,)),
        pltpu.SemaphoreType.DMA((_NSLOT,)),
        pltpu.SemaphoreType.DMA((4,)),
    ],
)
def _sc_scatter(m0, m1, src2d, dst2d, zblk, out, src_v, dst_v, rows_v, acc,
                gsem, ssem, isem):
    cid = lax.axis_index("c")
    sid = lax.axis_index("s")

    @pl.when(sid < _NSUB - 1)
    def _():
        pltpu.sync_copy(zblk.at[pl.ds(0, _S0)], acc.at[pl.ds(sid * _S0, _S0)])

    @pl.when(sid == _NSUB - 1)
    def _():
        pltpu.sync_copy(zblk.at[pl.ds(0, _S15)],
                        acc.at[pl.ds((_NSUB - 1) * _S0, _S15)])

    plsc.subcore_barrier()

    def run(m):
        rb0 = sid * (_EPT // _IB)
        pltpu.async_copy(src2d.at[pl.ds(rb0, _JN)], src_v.at[0], isem.at[0])
        pltpu.async_copy(dst2d.at[pl.ds(rb0, _JN)], dst_v.at[0], isem.at[1])

        def body(ci, carry):
            rb = rb0 + ci * _JN
            b = lax.rem(ci, 2)
            nb = 1 - b

            @pl.when(ci < _NCH - 1)
            def _():
                rbn = rb + _JN
                pltpu.async_copy(src2d.at[pl.ds(rbn, _JN)], src_v.at[nb],
                                 isem.at[2 * nb])
                pltpu.async_copy(dst2d.at[pl.ds(rbn, _JN)], dst_v.at[nb],
                                 isem.at[2 * nb + 1])

            pltpu.make_async_copy(src2d.at[pl.ds(rb, _JN)], src_v.at[b],
                                  isem.at[2 * b]).wait()
            pltpu.make_async_copy(dst2d.at[pl.ds(rb, _JN)], dst_v.at[b],
                                  isem.at[2 * b + 1]).wait()
            src_c = src_v.at[b]
            dst_c = dst_v.at[b]
            gd = [None] * _JN
            sd = [None] * _JN

            def fire_scatter(jj):
                gd[jj].wait()
                sd[jj] = pltpu.async_copy(
                    rows_v.at[jj % _NSLOT], acc.at[dst_c.at[jj]],
                    ssem.at[jj % _NSLOT], add=True)

            for j in range(_JN):
                slot = j % _NSLOT
                if j >= _NSLOT:
                    sd[j - _NSLOT].wait()
                gd[j] = pltpu.async_copy(
                    m.at[src_c.at[j]], rows_v.at[slot], gsem.at[slot])
                if j >= _NSLOT - 1:
                    fire_scatter(j - (_NSLOT - 1))
            for jj in range(_JN - (_NSLOT - 1), _JN):
                fire_scatter(jj)
            for jj in range(_JN - _NSLOT, _JN):
                sd[jj].wait()
            return carry
        lax.fori_loop(0, _NCH, body, 0)

    @pl.when(cid == 0)
    def _():
        run(m0)

    @pl.when(cid == 1)
    def _():
        run(m1)

    plsc.subcore_barrier()

    @pl.when(sid < _NSUB - 1)
    def _():
        pltpu.sync_copy(
            acc.at[pl.ds(sid * _S0, _S0)],
            out.at[pl.ds(cid * _N + sid * _S0, _S0)],
        )

    @pl.when(sid == _NSUB - 1)
    def _():
        pltpu.sync_copy(
            acc.at[pl.ds((_NSUB - 1) * _S0, _S15)],
            out.at[pl.ds(cid * _N + (_NSUB - 1) * _S0, _S15)],
        )


def kernel(x, W_in, b_in, msgW1, msgb1, msgW2, msgb2, updW1, updb1, updW2,
           updb2, gamma, beta, O1, o1, O2, o2, edge_index, n_qubits):
    src2d = edge_index[0].reshape(_E // _IB, _IB)
    dst2d = edge_index[1].reshape(_E // _IB, _IB)
    zblk = jnp.zeros((_S0, _HC), jnp.float32)
    h = _in_proj(x, W_in, b_in)
    for i in range(_L):
        m0, m1 = _msg(h, msgW1[i], msgb1[i], msgW2[i], msgb2[i])
        agg = _sc_scatter(m0, m1, src2d, dst2d, zblk)
        out_pre, st = _upd(h, agg[:_N], agg[_N:], updW1[i], updb1[i],
                           updW2[i], updb2[i])
        h = _norm(out_pre, h, st, gamma[i], beta[i])
    q = lax.dynamic_slice_in_dim(h, n_qubits - _NQ, _NQ, axis=0)
    return _read(q, O1, o1, O2, o2)


# speedup vs baseline: 9.8862x; 1.0840x over previous
import functools

import jax
import jax.numpy as jnp
from jax import lax
from jax.experimental import pallas as pl
from jax.experimental.pallas import tpu as pltpu
from jax.experimental.pallas import tpu_sc as plsc

_N = 50000
_E = 800000
_H = 64
_NQ = 25000
_L = 5

_BR = 2000
_BQ = 5000


def _rbf16(a):
    u = lax.bitcast_convert_type(a, jnp.uint32)
    u = u + jnp.uint32(0x7FFF) + ((u >> jnp.uint32(16)) & jnp.uint32(1))
    u = u & jnp.uint32(0xFFFF0000)
    return lax.bitcast_convert_type(u, jnp.float32)


def _dot(a, b):
    return jnp.dot(_rbf16(a), _rbf16(b), preferred_element_type=jnp.float32)


def _inproj_body(x_ref, w_ref, b_ref, o_ref):
    o_ref[...] = jnp.maximum(
        _dot(x_ref[...], w_ref[...])
        + b_ref[...], 0.0)


def _in_proj(x, W_in, b_in):
    return pl.pallas_call(
        _inproj_body,
        grid=(_N // _BR,),
        in_specs=[
            pl.BlockSpec((_BR, 3), lambda i: (i, 0)),
            pl.BlockSpec((3, _H), lambda i: (0, 0)),
            pl.BlockSpec((1, _H), lambda i: (0, 0)),
        ],
        out_specs=pl.BlockSpec((_BR, _H), lambda i: (i, 0)),
        out_shape=jax.ShapeDtypeStruct((_N, _H), jnp.float32),
    )(x, W_in, b_in.reshape(1, _H))


def _msg_body(h_ref, w1_ref, b1_ref, w2_ref, b2_ref, m0_ref, m1_ref):
    t = jnp.maximum(
        _dot(h_ref[...], w1_ref[...])
        + b1_ref[...], 0.0)
    m = _dot(t, w2_ref[...]) + b2_ref[...]
    m0_ref[...] = m[:, : _H // 2]
    m1_ref[...] = m[:, _H // 2:]


def _msg(h, w1, b1, w2, b2):
    return pl.pallas_call(
        _msg_body,
        grid=(_N // _BR,),
        in_specs=[
            pl.BlockSpec((_BR, _H), lambda i: (i, 0)),
            pl.BlockSpec((_H, _H), lambda i: (0, 0)),
            pl.BlockSpec((1, _H), lambda i: (0, 0)),
            pl.BlockSpec((_H, _H), lambda i: (0, 0)),
            pl.BlockSpec((1, _H), lambda i: (0, 0)),
        ],
        out_specs=[
            pl.BlockSpec((_BR, _H // 2), lambda i: (i, 0)),
            pl.BlockSpec((_BR, _H // 2), lambda i: (i, 0)),
        ],
        out_shape=[
            jax.ShapeDtypeStruct((_N, _H // 2), jnp.float32),
            jax.ShapeDtypeStruct((_N, _H // 2), jnp.float32),
        ],
    )(h, w1, b1.reshape(1, _H), w2, b2.reshape(1, _H))


def _upd_body(h_ref, a0_ref, a1_ref, w1_ref, b1_ref, w2_ref, b2_ref,
              o_ref, st_ref):
    comb = jnp.concatenate([h_ref[...], a0_ref[...], a1_ref[...]], axis=1)
    t = jnp.maximum(
        _dot(comb, w1_ref[...])
        + b1_ref[...], 0.0)
    o = _dot(t, w2_ref[...]) + b2_ref[...]
    o_ref[...] = o
    s = jnp.sum(o, axis=0, keepdims=True)
    s2 = jnp.sum(o * o, axis=0, keepdims=True)
    st = jnp.concatenate([s, s2], axis=0)

    @pl.when(pl.program_id(0) == 0)
    def _():
        st_ref[...] = st

    @pl.when(pl.program_id(0) != 0)
    def _():
        st_ref[...] += st


def _upd(h, agg, w1, b1, w2, b2):
    nblk = _N // _BR
    return pl.pallas_call(
        _upd_body,
        grid=(nblk,),
        in_specs=[
            pl.BlockSpec((_BR, _H), lambda i: (i, 0)),
            pl.BlockSpec((_BR, _H // 2), lambda i: (i, 0)),
            pl.BlockSpec((_BR, _H // 2), lambda i, n=nblk: (i + n, 0)),
            pl.BlockSpec((2 * _H, _H), lambda i: (0, 0)),
            pl.BlockSpec((1, _H), lambda i: (0, 0)),
            pl.BlockSpec((_H, _H), lambda i: (0, 0)),
            pl.BlockSpec((1, _H), lambda i: (0, 0)),
        ],
        out_specs=[
            pl.BlockSpec((_BR, _H), lambda i: (i, 0)),
            pl.BlockSpec((2, _H), lambda i: (0, 0)),
        ],
        out_shape=[
            jax.ShapeDtypeStruct((_N, _H), jnp.float32),
            jax.ShapeDtypeStruct((2, _H), jnp.float32),
        ],
    )(h, agg, agg, w1, b1.reshape(1, _H), w2, b2.reshape(1, _H))


def _norm_body(o_ref, h_ref, st_ref, g_ref, b_ref, out_ref):
    mean = st_ref[0:1, :] * (1.0 / _N)
    var = st_ref[1:2, :] * (1.0 / _N) - mean * mean
    inv = lax.rsqrt(var + 1e-5)
    out_ref[...] = jnp.maximum(
        (o_ref[...] - mean) * inv * g_ref[...] + b_ref[...] + h_ref[...], 0.0)


def _norm(o, h, st, g, b):
    return pl.pallas_call(
        _norm_body,
        grid=(_N // _BR,),
        in_specs=[
            pl.BlockSpec((_BR, _H), lambda i: (i, 0)),
            pl.BlockSpec((_BR, _H), lambda i: (i, 0)),
            pl.BlockSpec((2, _H), lambda i: (0, 0)),
            pl.BlockSpec((1, _H), lambda i: (0, 0)),
            pl.BlockSpec((1, _H), lambda i: (0, 0)),
        ],
        out_specs=pl.BlockSpec((_BR, _H), lambda i: (i, 0)),
        out_shape=jax.ShapeDtypeStruct((_N, _H), jnp.float32),
    )(o, h, st, g.reshape(1, _H), b.reshape(1, _H))


def _read_body(q_ref, o1_ref, b1_ref, o2_ref, b2_ref, out_ref):
    t = jnp.maximum(
        _dot(q_ref[...], o1_ref[...])
        + b1_ref[...], 0.0)
    out_ref[...] = (
        _dot(t, o2_ref[...]) + b2_ref[...])


def _read(q, O1, o1, O2, o2):
    return pl.pallas_call(
        _read_body,
        grid=(_NQ // _BQ,),
        in_specs=[
            pl.BlockSpec((_BQ, _H), lambda i: (i, 0)),
            pl.BlockSpec((_H, _H // 2), lambda i: (0, 0)),
            pl.BlockSpec((1, _H // 2), lambda i: (0, 0)),
            pl.BlockSpec((_H // 2, 1), lambda i: (0, 0)),
            pl.BlockSpec((1, 1), lambda i: (0, 0)),
        ],
        out_specs=pl.BlockSpec((_BQ, 1), lambda i: (i, 0)),
        out_shape=jax.ShapeDtypeStruct((_NQ, 1), jnp.float32),
    )(q, O1, o1.reshape(1, _H // 2), O2, o2.reshape(1, 1))


_NSUB = 16
_S0 = 3128
_S15 = _N - 15 * _S0
_EPT = _E // _NSUB
_IB = 125
_JN = 16
_CH = _IB * _JN
_NCH = _EPT // _CH
_HC = _H // 2
_NSLOT = 4

_sc_mesh = plsc.VectorSubcoreMesh(core_axis_name="c", subcore_axis_name="s")


@functools.partial(
    pl.kernel,
    out_type=jax.ShapeDtypeStruct((2 * _N, _HC), jnp.float32),
    mesh=_sc_mesh,
    compiler_params=pltpu.CompilerParams(use_tc_tiling_on_sc=False),
    scratch_types=[
        pltpu.VMEM((2, _JN, _IB), jnp.int32),
        pltpu.VMEM((2, _JN, _IB), jnp.int32),
        pltpu.VMEM((_NSLOT, _IB, _HC), jnp.float32),
        pltpu.VMEM_SHARED((_N, _HC), jnp.float32),
        pltpu.SemaphoreType.DMA((_NSLOT,)),
        pltpu.SemaphoreType.DMA((_NSLOT,)),
        pltpu.SemaphoreType.DMA((4,)),
    ],
)
def _sc_scatter(m0, m1, src2d, dst2d, zblk, out, src_v, dst_v, rows_v, acc,
                gsem, ssem, isem):
    cid = lax.axis_index("c")
    sid = lax.axis_index("s")

    @pl.when(sid < _NSUB - 1)
    def _():
        pltpu.sync_copy(zblk.at[pl.ds(0, _S0)], acc.at[pl.ds(sid * _S0, _S0)])

    @pl.when(sid == _NSUB - 1)
    def _():
        pltpu.sync_copy(zblk.at[pl.ds(0, _S15)],
                        acc.at[pl.ds((_NSUB - 1) * _S0, _S15)])

    plsc.subcore_barrier()

    def run(m):
        rb0 = sid * (_EPT // _IB)
        pltpu.async_copy(src2d.at[pl.ds(rb0, _JN)], src_v.at[0], isem.at[0])
        pltpu.async_copy(dst2d.at[pl.ds(rb0, _JN)], dst_v.at[0], isem.at[1])

        def body(ci, carry):
            rb = rb0 + ci * _JN
            b = lax.rem(ci, 2)
            nb = 1 - b

            @pl.when(ci < _NCH - 1)
            def _():
                rbn = rb + _JN
                pltpu.async_copy(src2d.at[pl.ds(rbn, _JN)], src_v.at[nb],
                                 isem.at[2 * nb])
                pltpu.async_copy(dst2d.at[pl.ds(rbn, _JN)], dst_v.at[nb],
                                 isem.at[2 * nb + 1])

            pltpu.make_async_copy(src2d.at[pl.ds(rb, _JN)], src_v.at[b],
                                  isem.at[2 * b]).wait()
            pltpu.make_async_copy(dst2d.at[pl.ds(rb, _JN)], dst_v.at[b],
                                  isem.at[2 * b + 1]).wait()
            src_c = src_v.at[b]
            dst_c = dst_v.at[b]
            gd = [None] * _JN
            sd = [None] * _JN

            def fire_scatter(jj):
                gd[jj].wait()
                sd[jj] = pltpu.async_copy(
                    rows_v.at[jj % _NSLOT], acc.at[dst_c.at[jj]],
                    ssem.at[jj % _NSLOT], add=True)

            for j in range(_JN):
                slot = j % _NSLOT
                if j >= _NSLOT:
                    sd[j - _NSLOT].wait()
                gd[j] = pltpu.async_copy(
                    m.at[src_c.at[j]], rows_v.at[slot], gsem.at[slot])
                if j >= _NSLOT - 1:
                    fire_scatter(j - (_NSLOT - 1))
            for jj in range(_JN - (_NSLOT - 1), _JN):
                fire_scatter(jj)
            for jj in range(_JN - _NSLOT, _JN):
                sd[jj].wait()
            return carry
        lax.fori_loop(0, _NCH, body, 0)

    @pl.when(cid == 0)
    def _():
        run(m0)

    @pl.when(cid == 1)
    def _():
        run(m1)

    plsc.subcore_barrier()

    @pl.when(sid < _NSUB - 1)
    def _():
        pltpu.sync_copy(
            acc.at[pl.ds(sid * _S0, _S0)],
            out.at[pl.ds(cid * _N + sid * _S0, _S0)],
        )

    @pl.when(sid == _NSUB - 1)
    def _():
        pltpu.sync_copy(
            acc.at[pl.ds((_NSUB - 1) * _S0, _S15)],
            out.at[pl.ds(cid * _N + (_NSUB - 1) * _S0, _S15)],
        )


def kernel(x, W_in, b_in, msgW1, msgb1, msgW2, msgb2, updW1, updb1, updW2,
           updb2, gamma, beta, O1, o1, O2, o2, edge_index, n_qubits):
    src2d = edge_index[0].reshape(_E // _IB, _IB)
    dst2d = edge_index[1].reshape(_E // _IB, _IB)
    zblk = jnp.zeros((_S0, _HC), jnp.float32)
    h = _in_proj(x, W_in, b_in)
    for i in range(_L):
        m0, m1 = _msg(h, msgW1[i], msgb1[i], msgW2[i], msgb2[i])
        agg = _sc_scatter(m0, m1, src2d, dst2d, zblk)
        out_pre, st = _upd(h, agg, updW1[i], updb1[i], updW2[i], updb2[i])
        h = _norm(out_pre, h, st, gamma[i], beta[i])
    q = lax.dynamic_slice_in_dim(h, n_qubits - _NQ, _NQ, axis=0)
    return _read(q, O1, o1, O2, o2)


# speedup vs baseline: 11.0812x; 1.1209x over previous
import functools

import jax
import jax.numpy as jnp
from jax import lax
from jax.experimental import pallas as pl
from jax.experimental.pallas import tpu as pltpu
from jax.experimental.pallas import tpu_sc as plsc

_N = 50000
_E = 800000
_H = 64
_NQ = 25000
_L = 5

_BR = 5000
_BQ = 5000


def _rbf16(a):
    u = lax.bitcast_convert_type(a, jnp.uint32)
    u = u + jnp.uint32(0x7FFF) + ((u >> jnp.uint32(16)) & jnp.uint32(1))
    u = u & jnp.uint32(0xFFFF0000)
    return lax.bitcast_convert_type(u, jnp.float32)


def _dot(a, b):
    return jnp.dot(_rbf16(a), _rbf16(b), preferred_element_type=jnp.float32)


def _inproj_body(x_ref, w_ref, b_ref, o_ref):
    o_ref[...] = jnp.maximum(
        _dot(x_ref[...], w_ref[...])
        + b_ref[...], 0.0)


def _in_proj(x, W_in, b_in):
    return pl.pallas_call(
        _inproj_body,
        grid=(_N // _BR,),
        in_specs=[
            pl.BlockSpec((_BR, 3), lambda i: (i, 0)),
            pl.BlockSpec((3, _H), lambda i: (0, 0)),
            pl.BlockSpec((1, _H), lambda i: (0, 0)),
        ],
        out_specs=pl.BlockSpec((_BR, _H), lambda i: (i, 0)),
        out_shape=jax.ShapeDtypeStruct((_N, _H), jnp.float32),
    )(x, W_in, b_in.reshape(1, _H))


def _msg_body(h_ref, w1_ref, b1_ref, w2_ref, b2_ref, m0_ref, m1_ref):
    t = jnp.maximum(
        _dot(h_ref[...], w1_ref[...])
        + b1_ref[...], 0.0)
    m = _dot(t, w2_ref[...]) + b2_ref[...]
    m0_ref[...] = m[:, : _H // 2]
    m1_ref[...] = m[:, _H // 2:]


def _msg(h, w1, b1, w2, b2):
    return pl.pallas_call(
        _msg_body,
        grid=(_N // _BR,),
        in_specs=[
            pl.BlockSpec((_BR, _H), lambda i: (i, 0)),
            pl.BlockSpec((_H, _H), lambda i: (0, 0)),
            pl.BlockSpec((1, _H), lambda i: (0, 0)),
            pl.BlockSpec((_H, _H), lambda i: (0, 0)),
            pl.BlockSpec((1, _H), lambda i: (0, 0)),
        ],
        out_specs=[
            pl.BlockSpec((_BR, _H // 2), lambda i: (i, 0)),
            pl.BlockSpec((_BR, _H // 2), lambda i: (i, 0)),
        ],
        out_shape=[
            jax.ShapeDtypeStruct((_N, _H // 2), jnp.float32),
            jax.ShapeDtypeStruct((_N, _H // 2), jnp.float32),
        ],
    )(h, w1, b1.reshape(1, _H), w2, b2.reshape(1, _H))


def _upd_body(h_ref, a0_ref, a1_ref, w1_ref, b1_ref, w2_ref, b2_ref,
              o_ref, st_ref):
    comb = jnp.concatenate([h_ref[...], a0_ref[...], a1_ref[...]], axis=1)
    t = jnp.maximum(
        _dot(comb, w1_ref[...])
        + b1_ref[...], 0.0)
    o = _dot(t, w2_ref[...]) + b2_ref[...]
    o_ref[...] = o
    s = jnp.sum(o, axis=0, keepdims=True)
    s2 = jnp.sum(o * o, axis=0, keepdims=True)
    st = jnp.concatenate([s, s2], axis=0)

    @pl.when(pl.program_id(0) == 0)
    def _():
        st_ref[...] = st

    @pl.when(pl.program_id(0) != 0)
    def _():
        st_ref[...] += st


def _upd(h, agg, w1, b1, w2, b2):
    nblk = _N // _BR
    return pl.pallas_call(
        _upd_body,
        grid=(nblk,),
        in_specs=[
            pl.BlockSpec((_BR, _H), lambda i: (i, 0)),
            pl.BlockSpec((_BR, _H // 2), lambda i: (i, 0)),
            pl.BlockSpec((_BR, _H // 2), lambda i, n=nblk: (i + n, 0)),
            pl.BlockSpec((2 * _H, _H), lambda i: (0, 0)),
            pl.BlockSpec((1, _H), lambda i: (0, 0)),
            pl.BlockSpec((_H, _H), lambda i: (0, 0)),
            pl.BlockSpec((1, _H), lambda i: (0, 0)),
        ],
        out_specs=[
            pl.BlockSpec((_BR, _H), lambda i: (i, 0)),
            pl.BlockSpec((2, _H), lambda i: (0, 0)),
        ],
        out_shape=[
            jax.ShapeDtypeStruct((_N, _H), jnp.float32),
            jax.ShapeDtypeStruct((2, _H), jnp.float32),
        ],
    )(h, agg, agg, w1, b1.reshape(1, _H), w2, b2.reshape(1, _H))


def _norm_body(o_ref, h_ref, st_ref, g_ref, b_ref, out_ref):
    mean = st_ref[0:1, :] * (1.0 / _N)
    var = st_ref[1:2, :] * (1.0 / _N) - mean * mean
    inv = lax.rsqrt(var + 1e-5)
    out_ref[...] = jnp.maximum(
        (o_ref[...] - mean) * inv * g_ref[...] + b_ref[...] + h_ref[...], 0.0)


def _norm_msg_body(o_ref, h_ref, st_ref, g_ref, b_ref, w1_ref, b1_ref,
                   w2_ref, b2_ref, hn_ref, m0_ref, m1_ref):
    mean = st_ref[0:1, :] * (1.0 / _N)
    var = st_ref[1:2, :] * (1.0 / _N) - mean * mean
    inv = lax.rsqrt(var + 1e-5)
    hn = jnp.maximum(
        (o_ref[...] - mean) * inv * g_ref[...] + b_ref[...] + h_ref[...], 0.0)
    hn_ref[...] = hn
    t = jnp.maximum(_dot(hn, w1_ref[...]) + b1_ref[...], 0.0)
    m = _dot(t, w2_ref[...]) + b2_ref[...]
    m0_ref[...] = m[:, : _H // 2]
    m1_ref[...] = m[:, _H // 2:]


def _norm_msg(o, h, st, g, b, w1, b1, w2, b2):
    return pl.pallas_call(
        _norm_msg_body,
        grid=(_N // _BR,),
        in_specs=[
            pl.BlockSpec((_BR, _H), lambda i: (i, 0)),
            pl.BlockSpec((_BR, _H), lambda i: (i, 0)),
            pl.BlockSpec((2, _H), lambda i: (0, 0)),
            pl.BlockSpec((1, _H), lambda i: (0, 0)),
            pl.BlockSpec((1, _H), lambda i: (0, 0)),
            pl.BlockSpec((_H, _H), lambda i: (0, 0)),
            pl.BlockSpec((1, _H), lambda i: (0, 0)),
            pl.BlockSpec((_H, _H), lambda i: (0, 0)),
            pl.BlockSpec((1, _H), lambda i: (0, 0)),
        ],
        out_specs=[
            pl.BlockSpec((_BR, _H), lambda i: (i, 0)),
            pl.BlockSpec((_BR, _H // 2), lambda i: (i, 0)),
            pl.BlockSpec((_BR, _H // 2), lambda i: (i, 0)),
        ],
        out_shape=[
            jax.ShapeDtypeStruct((_N, _H), jnp.float32),
            jax.ShapeDtypeStruct((_N, _H // 2), jnp.float32),
            jax.ShapeDtypeStruct((_N, _H // 2), jnp.float32),
        ],
    )(o, h, st, g.reshape(1, _H), b.reshape(1, _H), w1, b1.reshape(1, _H),
      w2, b2.reshape(1, _H))


def _norm(o, h, st, g, b):
    return pl.pallas_call(
        _norm_body,
        grid=(_N // _BR,),
        in_specs=[
            pl.BlockSpec((_BR, _H), lambda i: (i, 0)),
            pl.BlockSpec((_BR, _H), lambda i: (i, 0)),
            pl.BlockSpec((2, _H), lambda i: (0, 0)),
            pl.BlockSpec((1, _H), lambda i: (0, 0)),
            pl.BlockSpec((1, _H), lambda i: (0, 0)),
        ],
        out_specs=pl.BlockSpec((_BR, _H), lambda i: (i, 0)),
        out_shape=jax.ShapeDtypeStruct((_N, _H), jnp.float32),
    )(o, h, st, g.reshape(1, _H), b.reshape(1, _H))


def _read_body(q_ref, o1_ref, b1_ref, o2_ref, b2_ref, out_ref):
    t = jnp.maximum(
        _dot(q_ref[...], o1_ref[...])
        + b1_ref[...], 0.0)
    out_ref[...] = (
        _dot(t, o2_ref[...]) + b2_ref[...])


def _read(q, O1, o1, O2, o2):
    return pl.pallas_call(
        _read_body,
        grid=(_NQ // _BQ,),
        in_specs=[
            pl.BlockSpec((_BQ, _H), lambda i: (i, 0)),
            pl.BlockSpec((_H, _H // 2), lambda i: (0, 0)),
            pl.BlockSpec((1, _H // 2), lambda i: (0, 0)),
            pl.BlockSpec((_H // 2, 1), lambda i: (0, 0)),
            pl.BlockSpec((1, 1), lambda i: (0, 0)),
        ],
        out_specs=pl.BlockSpec((_BQ, 1), lambda i: (i, 0)),
        out_shape=jax.ShapeDtypeStruct((_NQ, 1), jnp.float32),
    )(q, O1, o1.reshape(1, _H // 2), O2, o2.reshape(1, 1))


_NSUB = 16
_S0 = 3128
_S15 = _N - 15 * _S0
_EPT = _E // _NSUB
_IB = 125
_JN = 16
_CH = _IB * _JN
_NCH = _EPT // _CH
_HC = _H // 2
_NSLOT = 5

_sc_mesh = plsc.VectorSubcoreMesh(core_axis_name="c", subcore_axis_name="s")


@functools.partial(
    pl.kernel,
    out_type=jax.ShapeDtypeStruct((2 * _N, _HC), jnp.float32),
    mesh=_sc_mesh,
    compiler_params=pltpu.CompilerParams(use_tc_tiling_on_sc=False),
    scratch_types=[
        pltpu.VMEM((2, _JN, _IB), jnp.int32),
        pltpu.VMEM((2, _JN, _IB), jnp.int32),
        pltpu.VMEM((_NSLOT, _IB, _HC), jnp.float32),
        pltpu.VMEM_SHARED((_N, _HC), jnp.float32),
        pltpu.SemaphoreType.DMA((_NSLOT,)),
        pltpu.SemaphoreType.DMA((_NSLOT,)),
        pltpu.SemaphoreType.DMA((4,)),
    ],
)
def _sc_scatter(m0, m1, src2d, dst2d, zblk, out, src_v, dst_v, rows_v, acc,
                gsem, ssem, isem):
    cid = lax.axis_index("c")
    sid = lax.axis_index("s")

    @pl.when(sid < _NSUB - 1)
    def _():
        pltpu.sync_copy(zblk.at[pl.ds(0, _S0)], acc.at[pl.ds(sid * _S0, _S0)])

    @pl.when(sid == _NSUB - 1)
    def _():
        pltpu.sync_copy(zblk.at[pl.ds(0, _S15)],
                        acc.at[pl.ds((_NSUB - 1) * _S0, _S15)])

    plsc.subcore_barrier()

    def run(m):
        rb0 = sid * (_EPT // _IB)
        pltpu.async_copy(src2d.at[pl.ds(rb0, _JN)], src_v.at[0], isem.at[0])
        pltpu.async_copy(dst2d.at[pl.ds(rb0, _JN)], dst_v.at[0], isem.at[1])

        def body(ci, carry):
            rb = rb0 + ci * _JN
            b = lax.rem(ci, 2)
            nb = 1 - b

            @pl.when(ci < _NCH - 1)
            def _():
                rbn = rb + _JN
                pltpu.async_copy(src2d.at[pl.ds(rbn, _JN)], src_v.at[nb],
                                 isem.at[2 * nb])
                pltpu.async_copy(dst2d.at[pl.ds(rbn, _JN)], dst_v.at[nb],
                                 isem.at[2 * nb + 1])

            pltpu.make_async_copy(src2d.at[pl.ds(rb, _JN)], src_v.at[b],
                                  isem.at[2 * b]).wait()
            pltpu.make_async_copy(dst2d.at[pl.ds(rb, _JN)], dst_v.at[b],
                                  isem.at[2 * b + 1]).wait()
            src_c = src_v.at[b]
            dst_c = dst_v.at[b]
            gd = [None] * _JN
            sd = [None] * _JN

            def fire_scatter(jj):
                gd[jj].wait()
                sd[jj] = pltpu.async_copy(
                    rows_v.at[jj % _NSLOT], acc.at[dst_c.at[jj]],
                    ssem.at[jj % _NSLOT], add=True)

            for j in range(_JN):
                slot = j % _NSLOT
                if j >= _NSLOT:
                    sd[j - _NSLOT].wait()
                gd[j] = pltpu.async_copy(
                    m.at[src_c.at[j]], rows_v.at[slot], gsem.at[slot])
                if j >= _NSLOT - 1:
                    fire_scatter(j - (_NSLOT - 1))
            for jj in range(_JN - (_NSLOT - 1), _JN):
                fire_scatter(jj)
            for jj in range(_JN - _NSLOT, _JN):
                sd[jj].wait()
            return carry
        lax.fori_loop(0, _NCH, body, 0)

    @pl.when(cid == 0)
    def _():
        run(m0)

    @pl.when(cid == 1)
    def _():
        run(m1)

    plsc.subcore_barrier()

    @pl.when(sid < _NSUB - 1)
    def _():
        pltpu.sync_copy(
            acc.at[pl.ds(sid * _S0, _S0)],
            out.at[pl.ds(cid * _N + sid * _S0, _S0)],
        )

    @pl.when(sid == _NSUB - 1)
    def _():
        pltpu.sync_copy(
            acc.at[pl.ds((_NSUB - 1) * _S0, _S15)],
            out.at[pl.ds(cid * _N + (_NSUB - 1) * _S0, _S15)],
        )


def kernel(x, W_in, b_in, msgW1, msgb1, msgW2, msgb2, updW1, updb1, updW2,
           updb2, gamma, beta, O1, o1, O2, o2, edge_index, n_qubits):
    src2d = edge_index[0].reshape(_E // _IB, _IB)
    dst2d = edge_index[1].reshape(_E // _IB, _IB)
    zblk = jnp.zeros((_S0, _HC), jnp.float32)
    h = _in_proj(x, W_in, b_in)
    m0, m1 = _msg(h, msgW1[0], msgb1[0], msgW2[0], msgb2[0])
    for i in range(_L):
        agg = _sc_scatter(m0, m1, src2d, dst2d, zblk)
        out_pre, st = _upd(h, agg, updW1[i], updb1[i], updW2[i], updb2[i])
        if i < _L - 1:
            h, m0, m1 = _norm_msg(out_pre, h, st, gamma[i], beta[i],
                                  msgW1[i + 1], msgb1[i + 1],
                                  msgW2[i + 1], msgb2[i + 1])
        else:
            h = _norm(out_pre, h, st, gamma[i], beta[i])
    q = lax.dynamic_slice_in_dim(h, n_qubits - _NQ, _NQ, axis=0)
    return _read(q, O1, o1, O2, o2)
